# baseline (speedup 1.0000x reference)
import functools
import math

import jax
import jax.numpy as jnp
from jax import lax
from jax.experimental import pallas as pl
from jax.experimental.pallas import tpu as pltpu
from jax.experimental.pallas import tpu_sc as plsc

_VOCAB = 256
_D = 2048
_NC = 2
_NS = 16
_NW = _NC * _NS
_LANES = 16
_CHUNK = 16
_NBUF = 2
_SCALE = math.sqrt(_D)
_FMT = plsc.PackFormat.INTERLEAVED


def _make_emb(n_tokens):
    bpw = n_tokens // _NW
    nchunk = bpw // _CHUNK
    rows_per_tile = _VOCAB // _NS
    groups = _D // (2 * _LANES)

    mesh = plsc.VectorSubcoreMesh(core_axis_name="c", subcore_axis_name="s")

    @functools.partial(
        pl.kernel,
        mesh=mesh,
        compiler_params=pltpu.CompilerParams(needs_layout_passes=False),
        out_type=[
            jax.ShapeDtypeStruct((n_tokens, _D), jnp.float32),
            jax.ShapeDtypeStruct((_NC, _VOCAB, _D // 2), jnp.int32),
        ],
        scratch_types=[
            pltpu.VMEM((nchunk, _CHUNK), jnp.int32),
            pltpu.VMEM((_NBUF, _CHUNK, _D // 2), jnp.int32),
            pltpu.VMEM((_NBUF, _CHUNK, _D), jnp.float32),
            pltpu.SemaphoreType.DMA,
            pltpu.SemaphoreType.DMA,
            pltpu.SemaphoreType.DMA,
            pltpu.SemaphoreType.DMA,
        ],
    )
    def emb(x_hbm, tab_hbm, out_hbm, tabscr_hbm, idx_v, ring_bf, ring_f,
            g0, g1, s0, s1):
        c = lax.axis_index("c")
        s = lax.axis_index("s")
        wid = s * _NC + c
        gsem = (g0, g1)
        ssem = (s0, s1)

        row0 = s * rows_per_tile
        stage = ring_f.at[0]
        bfstage = ring_bf.at[0]
        pltpu.sync_copy(tab_hbm.at[pl.ds(row0, rows_per_tile)], stage)

        def pack_row(r, carry):
            for j in range(groups):
                a = stage[r, pl.ds(j * 2 * _LANES, _LANES)] * _SCALE
                b = stage[r, pl.ds(j * 2 * _LANES + _LANES, _LANES)] * _SCALE
                packed = plsc.bitcast(plsc.pack(a, b, format=_FMT), jnp.int32)
                bfstage[r, pl.ds(j * _LANES, _LANES)] = packed
            return carry
        lax.fori_loop(0, rows_per_tile, pack_row, 0)

        @pl.when(s == 0)
        def _zero_row0():
            for j in range(groups):
                bfstage[0, pl.ds(j * _LANES, _LANES)] = jnp.zeros(
                    (_LANES,), jnp.int32)

        pltpu.sync_copy(bfstage, tabscr_hbm.at[c, pl.ds(row0, rows_per_tile)])
        plsc.subcore_barrier()

        pltpu.sync_copy(x_hbm.at[wid], idx_v)

        def gather(k, b):
            return pltpu.make_async_copy(
                tabscr_hbm.at[c].at[idx_v.at[k]], ring_bf.at[b], gsem[b])

        def scatter(k, b):
            base = wid * bpw + k * _CHUNK
            return pltpu.make_async_copy(
                ring_f.at[b], out_hbm.at[pl.ds(base, _CHUNK)], ssem[b])

        def expand(b):
            src = ring_bf.at[b]
            dst = ring_f.at[b]

            def row(r, carry):
                for j in range(groups):
                    v = plsc.bitcast(
                        src[r, pl.ds(j * _LANES, _LANES)], jnp.bfloat16)
                    lo, hi = plsc.unpack(v, format=_FMT)
                    dst[r, pl.ds(j * 2 * _LANES, _LANES)] = lo
                    dst[r, pl.ds(j * 2 * _LANES + _LANES, _LANES)] = hi
                return carry
            lax.fori_loop(0, _CHUNK, row, 0)

        for b in range(_NBUF):
            gather(b, b).start()

        def do_group(g, carry):
            for i in range(_NBUF):
                k = g * _NBUF + i
                b = i
                gather(k, b).wait()

                @pl.when(k >= _NBUF)
                def _drain_prev():
                    scatter(k - _NBUF, b).wait()
                expand(b)
                scatter(k, b).start()

                @pl.when(k + _NBUF < nchunk)
                def _prefetch():
                    gather(k + _NBUF, b).start()
            return carry
        lax.fori_loop(0, nchunk // _NBUF, do_group, 0)
        for b in range(_NBUF):
            scatter(nchunk - _NBUF + b, b).wait()

    return emb


def kernel(x, table):
    b, seq = x.shape
    n = b * seq
    x3 = x.astype(jnp.int32).reshape(_NW, n // (_NW * _CHUNK), _CHUNK)
    out, _ = _make_emb(n)(x3, table)
    return out.reshape(b, seq, _D)


# speedup vs baseline: 1.0333x; 1.0333x over previous
import functools
import math

import jax
import jax.numpy as jnp
from jax import lax
from jax.experimental import pallas as pl
from jax.experimental.pallas import tpu as pltpu
from jax.experimental.pallas import tpu_sc as plsc

_VOCAB = 256
_D = 2048
_NC = 2
_NS = 16
_NW = _NC * _NS
_LANES = 16
_CHUNK = 8
_NBUF = 4
_SCALE = math.sqrt(_D)


def _make_emb(n_tokens):
    bpw = n_tokens // _NW
    nchunk = bpw // _CHUNK
    rows_per_tile = _VOCAB // _NS

    mesh = plsc.VectorSubcoreMesh(core_axis_name="c", subcore_axis_name="s")

    @functools.partial(
        pl.kernel,
        mesh=mesh,
        out_type=[
            jax.ShapeDtypeStruct((n_tokens, _D), jnp.float32),
            jax.ShapeDtypeStruct((_NC, _VOCAB, _D), jnp.float32),
        ],
        scratch_types=[
            pltpu.VMEM((nchunk, _CHUNK), jnp.int32),
            pltpu.VMEM((_NBUF, _CHUNK, _D), jnp.float32),
            pltpu.VMEM((rows_per_tile, _D), jnp.float32),
            pltpu.SemaphoreType.DMA,
            pltpu.SemaphoreType.DMA,
            pltpu.SemaphoreType.DMA,
            pltpu.SemaphoreType.DMA,
            pltpu.SemaphoreType.DMA,
            pltpu.SemaphoreType.DMA,
            pltpu.SemaphoreType.DMA,
            pltpu.SemaphoreType.DMA,
        ],
    )
    def emb(x_hbm, tab_hbm, out_hbm, tabscr_hbm, idx_v, ring, stage,
            g0, g1, g2, g3, s0, s1, s2, s3):
        c = lax.axis_index("c")
        s = lax.axis_index("s")
        wid = s * _NC + c
        gsem = (g0, g1, g2, g3)
        ssem = (s0, s1, s2, s3)

        row0 = s * rows_per_tile
        pltpu.sync_copy(tab_hbm.at[pl.ds(row0, rows_per_tile)], stage)

        def scale_row(r, carry):
            for j in range(_D // _LANES):
                sl = pl.ds(j * _LANES, _LANES)
                stage[r, sl] = stage[r, sl] * _SCALE
            return carry
        lax.fori_loop(0, rows_per_tile, scale_row, 0)

        @pl.when(s == 0)
        def _zero_row0():
            for j in range(_D // _LANES):
                stage[0, pl.ds(j * _LANES, _LANES)] = jnp.zeros(
                    (_LANES,), jnp.float32)

        pltpu.sync_copy(stage, tabscr_hbm.at[c, pl.ds(row0, rows_per_tile)])
        plsc.subcore_barrier()

        pltpu.sync_copy(x_hbm.at[wid], idx_v)

        def gather(k, b):
            return pltpu.make_async_copy(
                tabscr_hbm.at[c].at[idx_v.at[k]], ring.at[b], gsem[b])

        def scatter(k, b):
            base = wid * bpw + k * _CHUNK
            return pltpu.make_async_copy(
                ring.at[b], out_hbm.at[pl.ds(base, _CHUNK)], ssem[b])

        for b in range(_NBUF - 1):
            gather(b, b).start()

        def do_group(g, carry):
            for i in range(_NBUF):
                k = g * _NBUF + i
                b = i
                b2 = (i + _NBUF - 1) % _NBUF
                gather(k, b).wait()
                scatter(k, b).start()

                @pl.when(k >= 1)
                def _drain_prev():
                    scatter(k - 1, b2).wait()

                @pl.when(k + _NBUF - 1 < nchunk)
                def _prefetch():
                    gather(k + _NBUF - 1, b2).start()
            return carry
        lax.fori_loop(0, nchunk // _NBUF, do_group, 0)
        scatter(nchunk - 1, (_NBUF - 1) % _NBUF).wait()

    return emb


_TC_BLK = 512


def _tc_emb(x2d, table):
    nblk = x2d.shape[0]

    def body(x_ref, tab_ref, o_ref):
        idx = x_ref[0, 0, :]
        iota = lax.broadcasted_iota(jnp.int32, (_TC_BLK, _VOCAB), 1)
        onehot = jnp.where(
            (idx[:, None] == iota) & (idx[:, None] != 0),
            jnp.float32(_SCALE), jnp.float32(0.0))
        o_ref[...] = jnp.dot(
            onehot, tab_ref[...], preferred_element_type=jnp.float32)

    return pl.pallas_call(
        body,
        grid=(nblk,),
        in_specs=[
            pl.BlockSpec((1, 1, _TC_BLK), lambda i: (i, 0, 0)),
            pl.BlockSpec((_VOCAB, _D), lambda i: (0, 0)),
        ],
        out_specs=pl.BlockSpec((_TC_BLK, _D), lambda i: (i, 0)),
        out_shape=jax.ShapeDtypeStruct((nblk * _TC_BLK, _D), jnp.float32),
    )(x2d, table)


def kernel(x, table):
    b, seq = x.shape
    n = b * seq
    half = n // 2
    xf = x.astype(jnp.int32).reshape(n)
    x_tc = xf[:half].reshape(half // _TC_BLK, 1, _TC_BLK)
    x_sc = xf[half:].reshape(_NW, half // (_NW * _CHUNK), _CHUNK)
    out_tc = _tc_emb(x_tc, table)
    out_sc, _ = _make_emb(half)(x_sc, table)
    out = jnp.concatenate([out_tc, out_sc], axis=0)
    return out.reshape(b, seq, _D)


# speedup vs baseline: 1.4491x; 1.4023x over previous
import functools
import math

import jax
import jax.numpy as jnp
from jax import lax
from jax.experimental import pallas as pl
from jax.experimental.pallas import tpu as pltpu
from jax.experimental.pallas import tpu_sc as plsc

_VOCAB = 256
_D = 2048
_NC = 2
_NS = 16
_NW = _NC * _NS
_LANES = 16
_CHUNK = 8
_NBUF = 4
_SCALE = math.sqrt(_D)


def _make_emb(n_tokens):
    bpw = n_tokens // _NW
    nchunk = bpw // _CHUNK
    rows_per_tile = _VOCAB // _NS

    mesh = plsc.VectorSubcoreMesh(core_axis_name="c", subcore_axis_name="s")

    @functools.partial(
        pl.kernel,
        mesh=mesh,
        out_type=[
            jax.ShapeDtypeStruct((n_tokens, _D), jnp.float32),
            jax.ShapeDtypeStruct((_NC, _VOCAB, _D), jnp.float32),
        ],
        scratch_types=[
            pltpu.VMEM((nchunk, _CHUNK), jnp.int32),
            pltpu.VMEM((_NBUF, _CHUNK, _D), jnp.float32),
            pltpu.VMEM((rows_per_tile, _D), jnp.float32),
            pltpu.SemaphoreType.DMA,
            pltpu.SemaphoreType.DMA,
            pltpu.SemaphoreType.DMA,
            pltpu.SemaphoreType.DMA,
            pltpu.SemaphoreType.DMA,
            pltpu.SemaphoreType.DMA,
            pltpu.SemaphoreType.DMA,
            pltpu.SemaphoreType.DMA,
        ],
    )
    def emb(x_hbm, tab_hbm, out_hbm, tabscr_hbm, idx_v, ring, stage,
            g0, g1, g2, g3, s0, s1, s2, s3):
        c = lax.axis_index("c")
        s = lax.axis_index("s")
        wid = s * _NC + c
        gsem = (g0, g1, g2, g3)
        ssem = (s0, s1, s2, s3)

        row0 = s * rows_per_tile
        pltpu.sync_copy(tab_hbm.at[pl.ds(row0, rows_per_tile)], stage)

        def scale_row(r, carry):
            for j in range(_D // _LANES):
                sl = pl.ds(j * _LANES, _LANES)
                stage[r, sl] = stage[r, sl] * _SCALE
            return carry
        lax.fori_loop(0, rows_per_tile, scale_row, 0)

        @pl.when(s == 0)
        def _zero_row0():
            for j in range(_D // _LANES):
                stage[0, pl.ds(j * _LANES, _LANES)] = jnp.zeros(
                    (_LANES,), jnp.float32)

        pltpu.sync_copy(stage, tabscr_hbm.at[c, pl.ds(row0, rows_per_tile)])
        plsc.subcore_barrier()

        pltpu.sync_copy(x_hbm.at[wid], idx_v)

        def gather(k, b):
            return pltpu.make_async_copy(
                tabscr_hbm.at[c].at[idx_v.at[k]], ring.at[b], gsem[b])

        def scatter(k, b):
            base = wid * bpw + k * _CHUNK
            return pltpu.make_async_copy(
                ring.at[b], out_hbm.at[pl.ds(base, _CHUNK)], ssem[b])

        for b in range(_NBUF - 1):
            gather(b, b).start()

        def do_group(g, carry):
            for i in range(_NBUF):
                k = g * _NBUF + i
                b = i
                b2 = (i + _NBUF - 1) % _NBUF
                gather(k, b).wait()
                scatter(k, b).start()

                @pl.when(k >= 1)
                def _drain_prev():
                    scatter(k - 1, b2).wait()

                @pl.when(k + _NBUF - 1 < nchunk)
                def _prefetch():
                    gather(k + _NBUF - 1, b2).start()
            return carry
        lax.fori_loop(0, nchunk // _NBUF, do_group, 0)
        scatter(nchunk - 1, (_NBUF - 1) % _NBUF).wait()

    return emb


def kernel(x, table):
    b, seq = x.shape
    n = b * seq
    x3 = x.astype(jnp.int32).reshape(_NW, n // (_NW * _CHUNK), _CHUNK)
    out, _ = _make_emb(n)(x3, table)
    return out.reshape(b, seq, _D)
